# baseline (speedup 1.0000x reference)
import jax
import jax.numpy as jnp
from jax.experimental import pallas as pl
from jax.experimental.pallas import tpu as pltpu


def _copy_body(x_hbm, e_hbm, x_out, e_out, xv, ev, s1, s2, s3, s4):
    c = pltpu.make_async_copy(x_hbm.at[pl.ds(0, 8), :], xv, s1)
    c.start()
    c.wait()
    o = pltpu.make_async_copy(xv, x_out.at[pl.ds(0, 8), :], s2)
    o.start()
    o.wait()
    ce = pltpu.make_async_copy(e_hbm.at[pl.ds(0, 8), :], ev, s3)
    ce.start()
    ce.wait()
    oe = pltpu.make_async_copy(ev, e_out.at[pl.ds(0, 8), :], s4)
    oe.start()
    oe.wait()


def kernel(x, x_lstm, encoded_z_gnss, edge_index, edge_attr):
    x_out, e_out = pl.pallas_call(
        _copy_body,
        out_shape=(
            jax.ShapeDtypeStruct(x.shape, x.dtype),
            jax.ShapeDtypeStruct(edge_attr.shape, edge_attr.dtype),
        ),
        in_specs=[
            pl.BlockSpec(memory_space=pl.ANY),
            pl.BlockSpec(memory_space=pl.ANY),
        ],
        out_specs=(
            pl.BlockSpec(memory_space=pl.ANY),
            pl.BlockSpec(memory_space=pl.ANY),
        ),
        scratch_shapes=[
            pltpu.MemorySpace.VMEM((8, 256), jnp.float32),
            pltpu.MemorySpace.VMEM((8, 16), jnp.float32),
            pltpu.SemaphoreType.DMA,
            pltpu.SemaphoreType.DMA,
            pltpu.SemaphoreType.DMA,
            pltpu.SemaphoreType.DMA,
        ],
    )(x, edge_attr)
    return (x_out, e_out)
